# baseline (speedup 1.0000x reference)
import functools

import jax
import jax.numpy as jnp
from jax import lax
from jax.experimental import pallas as pl
from jax.experimental.pallas import tpu as pltpu
from jax.experimental.pallas import tpu_sc as plsc

B = 4096
CONT = 13
NCAT = 26
VOCAB = 100000
D = 64

NC = 2
NS = 16
L = 16
NW = NC * NS
ROWS_PER_W = B * NCAT // NW
CHUNK = 128
NCHUNK = ROWS_PER_W // CHUNK
NBUF = 4


def _sc_gather_body(table_hbm, cat_hbm, out_hbm, idx_v, bufs_v, gsems, wsems):
    wid = lax.axis_index("s") * NC + lax.axis_index("c")
    row0 = wid * ROWS_PER_W

    pltpu.sync_copy(cat_hbm.at[pl.ds(wid * NCHUNK, NCHUNK), :], idx_v)

    def gather(j):
        b = j % NBUF
        return pltpu.make_async_copy(
            table_hbm.at[idx_v.at[j]], bufs_v.at[b], gsems.at[b])

    def writeback(j):
        b = j % NBUF
        return pltpu.make_async_copy(
            bufs_v.at[b], out_hbm.at[pl.ds(row0 + j * CHUNK, CHUNK), :],
            wsems.at[b])

    for j in range(NBUF):
        gather(j).start()
    for j in range(NCHUNK):
        gather(j).wait()
        writeback(j).start()
        if j + NBUF < NCHUNK:
            writeback(j).wait()
            gather(j + NBUF).start()
        else:
            writeback(j).wait()


def _sc_gather(table_flat, cat_rows):
    mesh = plsc.VectorSubcoreMesh(core_axis_name="c", subcore_axis_name="s")
    return pl.kernel(
        _sc_gather_body,
        out_type=jax.ShapeDtypeStruct((B * NCAT, D), jnp.float32),
        mesh=mesh,
        scratch_types=[
            pltpu.VMEM((NCHUNK, CHUNK), jnp.int32),
            pltpu.VMEM((NBUF, CHUNK, D), jnp.float32),
            pltpu.SemaphoreType.DMA((NBUF,)),
            pltpu.SemaphoreType.DMA((NBUF,)),
        ],
        compiler_params=pltpu.CompilerParams(use_tc_tiling_on_sc=False),
    )(table_flat, cat_rows)


def _tc_assemble_body(cont_ref, tab_ref, cat_ref, o_ref):
    cont_embed = cont_ref[...][:, :, None] * tab_ref[...][None, :, :]
    o_ref[...] = jnp.concatenate([cont_embed, cat_ref[...]], axis=1)


def _tc_assemble(continuous, cont_table, cat_embed):
    BB = 512
    grid = (B // BB,)
    return pl.pallas_call(
        _tc_assemble_body,
        grid=grid,
        in_specs=[
            pl.BlockSpec((BB, CONT), lambda i: (i, 0)),
            pl.BlockSpec((CONT, D), lambda i: (0, 0)),
            pl.BlockSpec((BB, NCAT, D), lambda i: (i, 0, 0)),
        ],
        out_specs=pl.BlockSpec((BB, CONT + NCAT, D), lambda i: (i, 0, 0)),
        out_shape=jax.ShapeDtypeStruct((B, CONT + NCAT, D), jnp.float32),
    )(continuous, cont_table, cat_embed)


@jax.jit
def kernel(continuous, categorical, cat_tables, cont_table):
    table_flat = cat_tables.reshape(NCAT * VOCAB, D)
    flat_idx = categorical + (jnp.arange(NCAT, dtype=jnp.int32) * VOCAB)[None, :]
    cat_rows = flat_idx.reshape(B * NCAT // CHUNK, CHUNK)
    gathered = _sc_gather(table_flat, cat_rows)
    cat_embed = gathered.reshape(B, NCAT, D)
    return _tc_assemble(continuous, cont_table, cat_embed)


# speedup vs baseline: 1.0006x; 1.0006x over previous
import functools

import jax
import jax.numpy as jnp
from jax import lax
from jax.experimental import pallas as pl
from jax.experimental.pallas import tpu as pltpu
from jax.experimental.pallas import tpu_sc as plsc

B = 4096
CONT = 13
NCAT = 26
VOCAB = 100000
D = 64

NC = 2
NS = 16
NW = NC * NS
BPW = B // NW
NBUF = 4


def _sc_gather_body(table_hbm, cat_hbm, out_hbm, idx_v, bufs_v, gsems, wsems):
    wid = lax.axis_index("s") * NC + lax.axis_index("c")
    b0 = wid * BPW

    pltpu.sync_copy(cat_hbm.at[:, pl.ds(b0, BPW)], idx_v)

    def gather(f):
        r = f % NBUF
        return pltpu.make_async_copy(
            table_hbm.at[f].at[idx_v.at[f]], bufs_v.at[r], gsems.at[r])

    def writeback(f):
        r = f % NBUF
        return pltpu.make_async_copy(
            bufs_v.at[r], out_hbm.at[pl.ds(b0, BPW), f, :], wsems.at[r])

    for f in range(NBUF):
        gather(f).start()
    for f in range(NCAT):
        gather(f).wait()
        writeback(f).start()
        writeback(f).wait()
        if f + NBUF < NCAT:
            gather(f + NBUF).start()


def _sc_gather(cat_tables, cat_t):
    mesh = plsc.VectorSubcoreMesh(core_axis_name="c", subcore_axis_name="s")
    return pl.kernel(
        _sc_gather_body,
        out_type=jax.ShapeDtypeStruct((B, NCAT, D), jnp.float32),
        mesh=mesh,
        scratch_types=[
            pltpu.VMEM((NCAT, BPW), jnp.int32),
            pltpu.VMEM((NBUF, BPW, D), jnp.float32),
            pltpu.SemaphoreType.DMA((NBUF,)),
            pltpu.SemaphoreType.DMA((NBUF,)),
        ],
        compiler_params=pltpu.CompilerParams(use_tc_tiling_on_sc=False),
    )(cat_tables, cat_t)


def _tc_assemble_body(cont_ref, tab_ref, cat_ref, o_ref):
    cont_embed = cont_ref[...][:, :, None] * tab_ref[...][None, :, :]
    o_ref[...] = jnp.concatenate([cont_embed, cat_ref[...]], axis=1)


def _tc_assemble(continuous, cont_table, cat_embed):
    BB = 256
    grid = (B // BB,)
    return pl.pallas_call(
        _tc_assemble_body,
        grid=grid,
        in_specs=[
            pl.BlockSpec((BB, CONT), lambda i: (i, 0)),
            pl.BlockSpec((CONT, D), lambda i: (0, 0)),
            pl.BlockSpec((BB, NCAT, D), lambda i: (i, 0, 0)),
        ],
        out_specs=pl.BlockSpec((BB, CONT + NCAT, D), lambda i: (i, 0, 0)),
        out_shape=jax.ShapeDtypeStruct((B, CONT + NCAT, D), jnp.float32),
    )(continuous, cont_table, cat_embed)


@jax.jit
def kernel(continuous, categorical, cat_tables, cont_table):
    cat_t = categorical.T
    cat_embed = _sc_gather(cat_tables, cat_t)
    return _tc_assemble(continuous, cont_table, cat_embed)


# speedup vs baseline: 1.2771x; 1.2764x over previous
import functools

import jax
import jax.numpy as jnp
from jax import lax
from jax.experimental import pallas as pl
from jax.experimental.pallas import tpu as pltpu
from jax.experimental.pallas import tpu_sc as plsc

B = 4096
CONT = 13
NCAT = 26
VOCAB = 100000
D = 64
L = 16

NC = 2
NS = 16
NW = NC * NS
CPAIRS = NCAT * D // NW
KPAIRS = CONT * D // NW
IB = B // 128
NBUF = 3


def _sc_body(table_hbm, catidx_hbm, cont_t_hbm, ctab_hbm, out_hbm,
             idx_v, ctab_v, bufs_v, cbufs_v, gsems, wsems, csems):
    wid = lax.axis_index("s") * NC + lax.axis_index("c")

    pltpu.sync_copy(ctab_hbm.at[pl.ds(0, CONT * D)], ctab_v)
    p0k = wid * KPAIRS
    for j in range(KPAIRS):
        p = p0k + j
        c = p // D
        d = p - c * D
        pltpu.sync_copy(cont_t_hbm.at[c], cbufs_v.at[j])
        ct = c * D + d
        g = (ct // L) * L
        lane = ct - g
        vals = ctab_v[pl.ds(g, L)]
        scal = vals.at[jnp.full((L,), lane, jnp.int32)].get(
            mode="promise_in_bounds")
        buf = cbufs_v.at[j]

        def body(s, _):
            buf[pl.ds(s * L, L)] = buf[pl.ds(s * L, L)] * scal
            return 0

        lax.fori_loop(0, B // L, body, 0)

    p0 = wid * CPAIRS
    f0 = p0 // D
    pltpu.sync_copy(catidx_hbm.at[pl.ds(f0, 2), :], idx_v)

    def gather(j):
        r = j % NBUF
        p = p0 + j
        f = p // D
        d = p - f * D
        return pltpu.make_async_copy(
            table_hbm.at[f, d].at[idx_v.at[f - f0]],
            bufs_v.at[r], gsems.at[r])

    def writeback(j):
        r = j % NBUF
        return pltpu.make_async_copy(
            bufs_v.at[r], out_hbm.at[CONT * D + p0 + j], wsems.at[r])

    for j in range(NBUF):
        gather(j).start()
    for j in range(CPAIRS):
        gather(j).wait()
        writeback(j).start()
        writeback(j).wait()
        if j + NBUF < CPAIRS:
            gather(j + NBUF).start()

    def cwriteback(j):
        return pltpu.make_async_copy(
            cbufs_v.at[j], out_hbm.at[p0k + j], csems.at[j % NBUF])

    for j in range(KPAIRS):
        cwriteback(j).start()
    for j in range(KPAIRS):
        cwriteback(j).wait()


def _sc_kernel(table_t, catidx, cont_t, ctab_flat):
    mesh = plsc.VectorSubcoreMesh(core_axis_name="c", subcore_axis_name="s")
    return pl.kernel(
        _sc_body,
        out_type=jax.ShapeDtypeStruct(((CONT + NCAT) * D, B), jnp.float32),
        mesh=mesh,
        scratch_types=[
            pltpu.VMEM((2, B), jnp.int32),
            pltpu.VMEM((CONT * D,), jnp.float32),
            pltpu.VMEM((NBUF, B), jnp.float32),
            pltpu.VMEM((KPAIRS, B), jnp.float32),
            pltpu.SemaphoreType.DMA((NBUF,)),
            pltpu.SemaphoreType.DMA((NBUF,)),
            pltpu.SemaphoreType.DMA((NBUF,)),
        ],
        compiler_params=pltpu.CompilerParams(use_tc_tiling_on_sc=False),
    )(table_t, catidx, cont_t, ctab_flat)


@jax.jit
def kernel(continuous, categorical, cat_tables, cont_table):
    table_t = jnp.transpose(cat_tables, (0, 2, 1))
    catidx = jnp.concatenate(
        [categorical.T, jnp.zeros((1, B), jnp.int32)], axis=0)
    cont_t = jnp.concatenate(
        [continuous.T, jnp.zeros((1, B), jnp.float32)], axis=0)
    ctab_flat = cont_table.reshape(CONT * D)
    out_t = _sc_kernel(table_t, catidx, cont_t, ctab_flat)
    return jnp.transpose(out_t.reshape(CONT + NCAT, D, B), (2, 0, 1))
